# baseline (speedup 1.0000x reference)
import functools

import jax
import jax.numpy as jnp
from jax.experimental import pallas as pl
from jax.experimental.pallas import tpu as pltpu

CODEBOOK = 1024
EMB = 256
B = 8
TOK = 1024
BPS = 2
STEPS = B // BPS
BETA = 0.25
N_TOKENS = B * TOK
N_ELEMS = N_TOKENS * EMB


def _vq_body(z_ref, w_ref,
             zq_ref, idx_ref, loss_ref, perp_ref, md_ref,
             z2sum_acc, sqsum_acc, counts_acc, zrs_acc):
    s = pl.program_id(0)
    w = w_ref[...]

    w2x = w + w
    w2 = jnp.sum(w * w, axis=1, keepdims=True)
    ci = jax.lax.broadcasted_iota(jnp.int32, (CODEBOOK, TOK), 0)
    big = jnp.int32(1 << 30)

    z2s_t = jnp.float32(0.0)
    sq_t = jnp.float32(0.0)
    cnt_t = None
    zrs_t = None
    for p in range(BPS):
        zb = z_ref[p]
        m2 = jax.lax.dot_general(w2x, zb, (((1,), (0,)), ((), ())),
                                 preferred_element_type=jnp.float32)
        z2 = jnp.sum(zb * zb, axis=0, keepdims=True)
        d = (z2 + w2) - m2

        minv = jnp.min(d, axis=0, keepdims=True)
        idx = jnp.min(jnp.where(d == minv, ci, big), axis=0, keepdims=True)
        idx_ref[p] = idx

        onehot = (ci == idx).astype(jnp.float32)
        zq_ref[p] = jax.lax.dot_general(w, onehot, (((0,), (0,)), ((), ())),
                                        preferred_element_type=jnp.float32)

        z2s_t = z2s_t + jnp.sum(z2)
        sq_t = sq_t + jnp.sum(minv)
        cnt_p = jnp.sum(onehot, axis=1, keepdims=True)
        zrs_p = jnp.sum(zb, axis=1, keepdims=True)
        cnt_t = cnt_p if cnt_t is None else cnt_t + cnt_p
        zrs_t = zrs_p if zrs_t is None else zrs_t + zrs_p

    @pl.when(s == 0)
    def _init():
        z2sum_acc[0, 0] = z2s_t
        sqsum_acc[0, 0] = sq_t
        counts_acc[...] = cnt_t
        zrs_acc[...] = zrs_t

    @pl.when(s > 0)
    def _acc():
        z2sum_acc[0, 0] += z2s_t
        sqsum_acc[0, 0] += sq_t
        counts_acc[...] += cnt_t
        zrs_acc[...] += zrs_t

    @pl.when(s == STEPS - 1)
    def _fin():
        wcs = jnp.sum(w2x, axis=0, keepdims=True)
        m2s = jax.lax.dot_general(wcs, zrs_acc[...], (((1,), (0,)), ((), ())),
                                  preferred_element_type=jnp.float32)
        dsum = (jnp.float32(CODEBOOK) * (z2sum_acc[0, 0]
                                         + jnp.float32(N_TOKENS) * jnp.sum(w2))
                - m2s[0, 0])
        md_ref[0, 0] = dsum / jnp.float32(N_TOKENS * CODEBOOK)
        msq = sqsum_acc[0, 0] / jnp.float32(N_ELEMS)
        loss_ref[0, 0] = jnp.float32(BETA) * msq + msq
        e = counts_acc[...] / jnp.float32(N_TOKENS)
        ent = jnp.sum(e * jnp.log(e + jnp.float32(1e-10)))
        perp_ref[0, 0] = jnp.exp(-ent)


@functools.partial(jax.jit, static_argnames=("interpret",))
def kernel(z, W, interpret=False):
    z3 = z.reshape(B, EMB, TOK)
    grid = (STEPS,)
    out_shapes = (
        jax.ShapeDtypeStruct((B, EMB, TOK), jnp.float32),
        jax.ShapeDtypeStruct((B, 1, TOK), jnp.int32),
        jax.ShapeDtypeStruct((1, 1), jnp.float32),
        jax.ShapeDtypeStruct((1, 1), jnp.float32),
        jax.ShapeDtypeStruct((1, 1), jnp.float32),
    )
    zq, idx, loss, perp, md = pl.pallas_call(
        _vq_body,
        grid=grid,
        in_specs=[
            pl.BlockSpec((BPS, EMB, TOK), lambda s: (s, 0, 0)),
            pl.BlockSpec((CODEBOOK, EMB), lambda s: (0, 0)),
        ],
        out_specs=(
            pl.BlockSpec((BPS, EMB, TOK), lambda s: (s, 0, 0)),
            pl.BlockSpec((BPS, 1, TOK), lambda s: (s, 0, 0)),
            pl.BlockSpec(memory_space=pltpu.SMEM),
            pl.BlockSpec(memory_space=pltpu.SMEM),
            pl.BlockSpec(memory_space=pltpu.SMEM),
        ),
        out_shape=out_shapes,
        scratch_shapes=[
            pltpu.SMEM((1, 1), jnp.float32),
            pltpu.SMEM((1, 1), jnp.float32),
            pltpu.VMEM((CODEBOOK, 1), jnp.float32),
            pltpu.VMEM((EMB, 1), jnp.float32),
        ],
        interpret=interpret,
    )(z3, W)

    z_q = zq.reshape(B, EMB, 32, 32)
    min_encoding_indices = idx.reshape(N_TOKENS, 1)
    return (z_q, loss[0, 0], perp[0, 0], md[0, 0], min_encoding_indices)
